# baseline (speedup 1.0000x reference)
import jax
import jax.numpy as jnp
from jax.experimental import pallas as pl
from jax.experimental.pallas import tpu as pltpu

B = 64
HW = 224
C_IN = 3
C_CONV = 64
N_CLS = 8
R_OUT = 8

RB = HW // 2
QB = HW // 4
MROWS = RB * QB
PIX = HW * HW
LROW = 228 * C_IN
LPAIR = 2 * LROW


def _conv_gap_kernel(x_ref, w_ref, b_ref, o_ref, p3_ref):
    f32 = jnp.float32
    bf16 = jnp.bfloat16
    QCH = 14
    total = jnp.zeros((1, 512), dtype=f32)
    for ck in range(QB // QCH):
        for qi in range(QCH):
            q = ck * QCH + qi
            c0 = 12 * q
            pieces = []
            for rsh in (0, 1):
                for par in (0, 1):
                    lo = par * LROW + c0
                    pieces.append(x_ref[0, rsh:rsh + RB, lo:lo + 18])
            pf = jnp.concatenate(pieces, axis=1)
            ph = pf.astype(bf16)
            plo = (pf - ph.astype(f32)).astype(bf16)
            p3_ref[112 * q: 112 * (q + 1), :] = jnp.concatenate(
                [ph, ph, plo], axis=1)
        m0 = ck * QCH * RB
        y = jax.lax.dot_general(
            p3_ref[m0: m0 + QCH * RB, :], w_ref[...],
            dimension_numbers=(((1,), (0,)), ((), ())),
            preferred_element_type=f32)
        y = jnp.maximum(y + b_ref[0:1, :], 0.0)
        total = total + jnp.sum(y, axis=0, keepdims=True)
    o_ref[0, 0, :] = total[0, :]


def _head_kernel(f_ref, wc_ref, bc_ref, wr_ref, br_ref, p_ref, r_ref):
    f5 = f_ref[...]
    f = (f5[:, 0:64] + f5[:, 64:128] + f5[:, 128:192] + f5[:, 192:256]
         + f5[:, 256:320] + f5[:, 320:384] + f5[:, 384:448]
         + f5[:, 448:512]) * (1.0 / PIX)
    logits = jax.lax.dot_general(
        f, wc_ref[...], dimension_numbers=(((1,), (0,)), ((), ())),
        preferred_element_type=jnp.float32) + bc_ref[0:1, :]
    mx = jnp.max(logits, axis=1, keepdims=True)
    e = jnp.exp(logits - mx)
    probs = e / jnp.sum(e, axis=1, keepdims=True)
    pmx = jnp.max(probs, axis=1, keepdims=True)
    lane = jax.lax.broadcasted_iota(jnp.int32, (B, N_CLS), 1)
    cand = jnp.where(probs == pmx, lane, N_CLS)
    amin = jnp.min(cand, axis=1, keepdims=True)
    onehot = (lane == amin).astype(jnp.float32)
    allr = jax.lax.dot_general(
        f, wr_ref[...], dimension_numbers=(((1,), (0,)), ((), ())),
        preferred_element_type=jnp.float32) + br_ref[0:1, :]
    ei = jax.lax.broadcasted_iota(jnp.int32, (N_CLS, N_CLS * R_OUT), 0)
    ki = jax.lax.broadcasted_iota(jnp.int32, (N_CLS, N_CLS * R_OUT), 1)
    rep = (ki // R_OUT == ei).astype(jnp.float32)
    mask = jax.lax.dot_general(
        onehot, rep, dimension_numbers=(((1,), (0,)), ((), ())),
        preferred_element_type=jnp.float32)
    sel = allr * mask
    routed = (sel[:, 0:8] + sel[:, 8:16] + sel[:, 16:24] + sel[:, 24:32]
              + sel[:, 32:40] + sel[:, 40:48] + sel[:, 48:56] + sel[:, 56:64])
    p_ref[...] = probs
    r_ref[...] = routed


def kernel(inputs, W_conv, b_conv, W_cls, b_cls, W_reg, b_reg):
    f32 = jnp.float32
    xp = jnp.pad(inputs, ((0, 0), (1, 3), (1, 3), (0, 0)))
    xrp = xp.reshape(B, 114, LPAIR)
    a = jnp.arange(2)
    s = jnp.arange(4)
    dr = jnp.arange(4)
    u = jnp.arange(6)
    dy = dr[:, None] - a[None, :]
    dx = u[:, None] - s[None, :]
    vy = ((dy >= 0) & (dy <= 2)).astype(f32)
    vx = ((dx >= 0) & (dx <= 2)).astype(f32)
    wt = W_conv[jnp.clip(dy, 0, 2)]
    wt = wt[:, :, jnp.clip(dx, 0, 2)]
    wt = (wt * vy[:, :, None, None, None, None]
          * vx[None, None, :, :, None, None])
    wt = wt.transpose(0, 2, 4, 1, 3, 5)
    wg = wt.reshape(72, 512)
    whi = wg.astype(jnp.bfloat16)
    wlo = (wg - whi.astype(f32)).astype(jnp.bfloat16)
    w3 = jnp.concatenate([whi, wlo, whi], axis=0)
    b512 = jnp.tile(b_conv, 8).reshape(1, 512)
    wrf = W_reg.transpose(1, 0, 2).reshape(C_CONV, N_CLS * R_OUT)
    brf = b_reg.reshape(1, N_CLS * R_OUT)
    bcf = b_cls.reshape(1, N_CLS)

    feat512 = pl.pallas_call(
        _conv_gap_kernel,
        grid=(B,),
        in_specs=[
            pl.BlockSpec((1, 114, LPAIR), lambda b: (b, 0, 0)),
            pl.BlockSpec((216, 512), lambda b: (0, 0)),
            pl.BlockSpec((1, 512), lambda b: (0, 0)),
        ],
        out_specs=pl.BlockSpec((1, 1, 512), lambda b: (b, 0, 0)),
        out_shape=jax.ShapeDtypeStruct((B, 1, 512), f32),
        scratch_shapes=[pltpu.VMEM((MROWS, 216), jnp.bfloat16)],
    )(xrp, w3, b512)
    feat512 = feat512.reshape(B, 512)

    probs, routed = pl.pallas_call(
        _head_kernel,
        in_specs=[
            pl.BlockSpec((B, 512), lambda: (0, 0)),
            pl.BlockSpec((C_CONV, N_CLS), lambda: (0, 0)),
            pl.BlockSpec((1, N_CLS), lambda: (0, 0)),
            pl.BlockSpec((C_CONV, N_CLS * R_OUT), lambda: (0, 0)),
            pl.BlockSpec((1, N_CLS * R_OUT), lambda: (0, 0)),
        ],
        out_specs=[
            pl.BlockSpec((B, N_CLS), lambda: (0, 0)),
            pl.BlockSpec((B, R_OUT), lambda: (0, 0)),
        ],
        out_shape=[
            jax.ShapeDtypeStruct((B, N_CLS), f32),
            jax.ShapeDtypeStruct((B, R_OUT), f32),
        ],
    )(feat512, W_cls, bcf, wrf, brf)
    return (probs, routed)


# speedup vs baseline: 1.0574x; 1.0574x over previous
import jax
import jax.numpy as jnp
from jax.experimental import pallas as pl
from jax.experimental.pallas import tpu as pltpu

B = 64
HW = 224
C_IN = 3
C_CONV = 64
N_CLS = 8
R_OUT = 8

RB = HW // 2
QB = HW // 4
MROWS = RB * QB
PIX = HW * HW
LROW = 228 * C_IN
LPAIR = 2 * LROW


def _conv_gap_kernel(x_ref, w_ref, b_ref, o_ref, xs_ref, p3_ref):
    f32 = jnp.float32
    bf16 = jnp.bfloat16
    QCH = 14
    z1 = jnp.zeros((1, 672), dtype=f32)
    z2 = jnp.zeros((2, 672), dtype=f32)
    zc3 = jnp.zeros((114, 3), dtype=f32)
    zc9 = jnp.zeros((114, 9), dtype=f32)
    par0 = jnp.concatenate([z1, x_ref[0, 0:112, 672:1344], z1], axis=0)
    par1 = jnp.concatenate([x_ref[0, 0:112, 0:672], z2], axis=0)
    xs_ref[...] = jnp.concatenate(
        [zc3, par0, zc9, zc3, par1, zc9], axis=1)
    total = jnp.zeros((1, 512), dtype=f32)
    for ck in range(QB // QCH):
        for qi in range(QCH):
            q = ck * QCH + qi
            c0 = 12 * q
            pieces = []
            for rsh in (0, 1):
                for par in (0, 1):
                    lo = par * LROW + c0
                    pieces.append(xs_ref[rsh:rsh + RB, lo:lo + 18])
            pf = jnp.concatenate(pieces, axis=1)
            ph = pf.astype(bf16)
            plo = (pf - ph.astype(f32)).astype(bf16)
            p3_ref[112 * q: 112 * (q + 1), :] = jnp.concatenate(
                [ph, ph, plo], axis=1)
        m0 = ck * QCH * RB
        y = jax.lax.dot_general(
            p3_ref[m0: m0 + QCH * RB, :], w_ref[...],
            dimension_numbers=(((1,), (0,)), ((), ())),
            preferred_element_type=f32)
        y = jnp.maximum(y + b_ref[0:1, :], 0.0)
        total = total + jnp.sum(y, axis=0, keepdims=True)
    o_ref[0, 0, :] = total[0, :]


def _head_kernel(f_ref, wc_ref, bc_ref, wr_ref, br_ref, p_ref, r_ref):
    f5 = f_ref[...]
    f = (f5[:, 0:64] + f5[:, 64:128] + f5[:, 128:192] + f5[:, 192:256]
         + f5[:, 256:320] + f5[:, 320:384] + f5[:, 384:448]
         + f5[:, 448:512]) * (1.0 / PIX)
    logits = jax.lax.dot_general(
        f, wc_ref[...], dimension_numbers=(((1,), (0,)), ((), ())),
        preferred_element_type=jnp.float32) + bc_ref[0:1, :]
    mx = jnp.max(logits, axis=1, keepdims=True)
    e = jnp.exp(logits - mx)
    probs = e / jnp.sum(e, axis=1, keepdims=True)
    pmx = jnp.max(probs, axis=1, keepdims=True)
    lane = jax.lax.broadcasted_iota(jnp.int32, (B, N_CLS), 1)
    cand = jnp.where(probs == pmx, lane, N_CLS)
    amin = jnp.min(cand, axis=1, keepdims=True)
    onehot = (lane == amin).astype(jnp.float32)
    allr = jax.lax.dot_general(
        f, wr_ref[...], dimension_numbers=(((1,), (0,)), ((), ())),
        preferred_element_type=jnp.float32) + br_ref[0:1, :]
    ei = jax.lax.broadcasted_iota(jnp.int32, (N_CLS, N_CLS * R_OUT), 0)
    ki = jax.lax.broadcasted_iota(jnp.int32, (N_CLS, N_CLS * R_OUT), 1)
    rep = (ki // R_OUT == ei).astype(jnp.float32)
    mask = jax.lax.dot_general(
        onehot, rep, dimension_numbers=(((1,), (0,)), ((), ())),
        preferred_element_type=jnp.float32)
    sel = allr * mask
    routed = (sel[:, 0:8] + sel[:, 8:16] + sel[:, 16:24] + sel[:, 24:32]
              + sel[:, 32:40] + sel[:, 40:48] + sel[:, 48:56] + sel[:, 56:64])
    p_ref[...] = probs
    r_ref[...] = routed


def kernel(inputs, W_conv, b_conv, W_cls, b_cls, W_reg, b_reg):
    f32 = jnp.float32
    xrp = inputs.reshape(B, 112, 1344)
    a = jnp.arange(2)
    s = jnp.arange(4)
    dr = jnp.arange(4)
    u = jnp.arange(6)
    dy = dr[:, None] - a[None, :]
    dx = u[:, None] - s[None, :]
    vy = ((dy >= 0) & (dy <= 2)).astype(f32)
    vx = ((dx >= 0) & (dx <= 2)).astype(f32)
    wt = W_conv[jnp.clip(dy, 0, 2)]
    wt = wt[:, :, jnp.clip(dx, 0, 2)]
    wt = (wt * vy[:, :, None, None, None, None]
          * vx[None, None, :, :, None, None])
    wt = wt.transpose(0, 2, 4, 1, 3, 5)
    wg = wt.reshape(72, 512)
    whi = wg.astype(jnp.bfloat16)
    wlo = (wg - whi.astype(f32)).astype(jnp.bfloat16)
    w3 = jnp.concatenate([whi, wlo, whi], axis=0)
    b512 = jnp.tile(b_conv, 8).reshape(1, 512)
    wrf = W_reg.transpose(1, 0, 2).reshape(C_CONV, N_CLS * R_OUT)
    brf = b_reg.reshape(1, N_CLS * R_OUT)
    bcf = b_cls.reshape(1, N_CLS)

    feat512 = pl.pallas_call(
        _conv_gap_kernel,
        grid=(B,),
        in_specs=[
            pl.BlockSpec((1, 112, 1344), lambda b: (b, 0, 0)),
            pl.BlockSpec((216, 512), lambda b: (0, 0)),
            pl.BlockSpec((1, 512), lambda b: (0, 0)),
        ],
        out_specs=pl.BlockSpec((1, 1, 512), lambda b: (b, 0, 0)),
        out_shape=jax.ShapeDtypeStruct((B, 1, 512), f32),
        scratch_shapes=[pltpu.VMEM((114, LPAIR), jnp.float32),
                        pltpu.VMEM((MROWS, 216), jnp.bfloat16)],
    )(xrp, w3, b512)
    feat512 = feat512.reshape(B, 512)

    probs, routed = pl.pallas_call(
        _head_kernel,
        in_specs=[
            pl.BlockSpec((B, 512), lambda: (0, 0)),
            pl.BlockSpec((C_CONV, N_CLS), lambda: (0, 0)),
            pl.BlockSpec((1, N_CLS), lambda: (0, 0)),
            pl.BlockSpec((C_CONV, N_CLS * R_OUT), lambda: (0, 0)),
            pl.BlockSpec((1, N_CLS * R_OUT), lambda: (0, 0)),
        ],
        out_specs=[
            pl.BlockSpec((B, N_CLS), lambda: (0, 0)),
            pl.BlockSpec((B, R_OUT), lambda: (0, 0)),
        ],
        out_shape=[
            jax.ShapeDtypeStruct((B, N_CLS), f32),
            jax.ShapeDtypeStruct((B, R_OUT), f32),
        ],
    )(feat512, W_cls, bcf, wrf, brf)
    return (probs, routed)
